# Initial kernel scaffold; baseline (speedup 1.0000x reference)
import jax
import jax.numpy as jnp
from jax.experimental import pallas as pl


def kernel(x_sites, x_wells, edge_index_s2w, edge_index_w2s, Wl_s2w, bl_s2w, Wr_s2w, Wl_w2s, bl_w2s, Wr_w2s):
    raise NotImplementedError("write your pallas kernel here")


# speedup vs baseline: 3.3653x; 3.3653x over previous
import functools

import jax
import jax.numpy as jnp
from jax import lax
from jax.experimental import pallas as pl
from jax.experimental.pallas import tpu as pltpu
from jax.experimental.pallas import tpu_sc as plsc

N = 10000
D = 256
H = 128
E = 160000
K = 128
NCHUNK = E // K
NC = 2
NS = 16
R1 = 624
TAIL = N - NS * R1
MAXJ = -(-NCHUNK // NS)


def _sc_segsum(tab_a, src_a, dst_a, tab_b, src_b, dst_b):
  mesh = plsc.VectorSubcoreMesh(
      core_axis_name="c", subcore_axis_name="s", num_cores=NC,
      num_subcores=NS)

  @functools.partial(
      pl.kernel,
      out_type=[
          jax.ShapeDtypeStruct((NC, N, H), jnp.float32),
          jax.ShapeDtypeStruct((NC, N, H), jnp.float32),
          jax.ShapeDtypeStruct((N, H), jnp.float32),
          jax.ShapeDtypeStruct((N, H), jnp.float32),
      ],
      mesh=mesh,
      scratch_types=[
          pltpu.VMEM((K,), jnp.int32),
          pltpu.VMEM((K,), jnp.int32),
          pltpu.VMEM((K, H), jnp.float32),
          pltpu.VMEM_SHARED((N, H), jnp.float32),
          pltpu.SemaphoreType.DMA,
      ],
  )
  def seg_kernel(taba_h, srca_h, dsta_h, tabb_h, srcb_h, dstb_h, zrows_h,
                 ones_hh, acca_out, accb_out, dega_out, degb_out,
                 idxs_v, idxd_v, rows_v, acc_sh, sem):
    c = lax.axis_index("c")
    s = lax.axis_index("s")
    r0 = s * R1
    last = s == NS - 1
    nfull = R1 // K
    rem = R1 - nfull * K

    def clear_accumulators():
      def clr(j, carry):
        pltpu.sync_copy(rows_v, acc_sh.at[pl.ds(r0 + j * K, K)])
        return carry

      lax.fori_loop(0, nfull, clr, 0)
      pltpu.sync_copy(rows_v.at[pl.ds(0, rem)],
                      acc_sh.at[pl.ds(r0 + R1 - rem, rem)])

      @pl.when(last)
      def _():
        pltpu.sync_copy(rows_v.at[pl.ds(0, TAIL)],
                        acc_sh.at[pl.ds(NS * R1, TAIL)])

    def run_edges(tab_h, src_h, dst_h):
      def chunk_body(j, carry):
        cid = j * NS + s

        @pl.when(cid < NCHUNK)
        def _():
          base = cid * K
          pltpu.sync_copy(src_h.at[pl.ds(base, K)], idxs_v)
          pltpu.sync_copy(dst_h.at[pl.ds(base, K)], idxd_v)
          pltpu.async_copy(tab_h.at[c].at[idxs_v], rows_v, sem).wait()
          pltpu.sync_copy(rows_v, acc_sh.at[idxd_v], add=True)

        return carry

      lax.fori_loop(0, MAXJ, chunk_body, 0)

    def write_back(get_dst):
      def wb(j, carry):
        pltpu.sync_copy(acc_sh.at[pl.ds(r0 + j * K, K)], rows_v)
        pltpu.sync_copy(rows_v, get_dst(r0 + j * K, K))
        return carry

      lax.fori_loop(0, nfull, wb, 0)
      pltpu.sync_copy(acc_sh.at[pl.ds(r0 + R1 - rem, rem)],
                      rows_v.at[pl.ds(0, rem)])
      pltpu.sync_copy(rows_v.at[pl.ds(0, rem)],
                      get_dst(r0 + R1 - rem, rem))

      @pl.when(last)
      def _():
        pltpu.sync_copy(acc_sh.at[pl.ds(NS * R1, TAIL)],
                        rows_v.at[pl.ds(rem, TAIL)])
        pltpu.sync_copy(rows_v.at[pl.ds(rem, TAIL)],
                        get_dst(NS * R1, TAIL))

    def run_deg(dst_h):
      def chunk_body(j, carry):
        cid = j * NS + s

        @pl.when(cid < NCHUNK)
        def _():
          pltpu.sync_copy(dst_h.at[pl.ds(cid * K, K)], idxd_v)
          pltpu.sync_copy(rows_v, acc_sh.at[idxd_v], add=True)

        return carry

      lax.fori_loop(0, MAXJ, chunk_body, 0)

    pltpu.sync_copy(zrows_h, rows_v)
    clear_accumulators()
    plsc.subcore_barrier()
    run_edges(taba_h, srca_h, dsta_h)
    plsc.subcore_barrier()
    write_back(lambda rr, nn: acca_out.at[c, pl.ds(rr, nn)])
    pltpu.sync_copy(zrows_h, rows_v)
    clear_accumulators()
    plsc.subcore_barrier()
    run_edges(tabb_h, srcb_h, dstb_h)
    plsc.subcore_barrier()
    write_back(lambda rr, nn: accb_out.at[c, pl.ds(rr, nn)])
    pltpu.sync_copy(zrows_h, rows_v)
    clear_accumulators()
    plsc.subcore_barrier()
    pltpu.sync_copy(ones_hh, rows_v)

    @pl.when(c == 0)
    def _():
      run_deg(dsta_h)

    @pl.when(c == 1)
    def _():
      run_deg(dstb_h)

    plsc.subcore_barrier()

    @pl.when(c == 0)
    def _():
      write_back(lambda rr, nn: dega_out.at[pl.ds(rr, nn)])

    @pl.when(c == 1)
    def _():
      write_back(lambda rr, nn: degb_out.at[pl.ds(rr, nn)])

  return seg_kernel(tab_a, src_a, dst_a, tab_b, src_b, dst_b,
                    jnp.zeros((K, H), jnp.float32),
                    jnp.ones((K, H), jnp.float32))


BM = 1000


def _tc_combine_body(acc0_ref, acc1_ref, deg_ref, x_ref, wl_ref, bl_ref,
                     wr_ref, o_ref):
  a = jnp.concatenate([acc0_ref[...], acc1_ref[...]], axis=1)
  d = jnp.maximum(deg_ref[:, 0:1], 1.0)
  z = lax.dot(a, wl_ref[...], preferred_element_type=jnp.float32) / d
  z = z + bl_ref[...] + lax.dot(x_ref[...], wr_ref[...],
                                preferred_element_type=jnp.float32)
  o_ref[...] = z


def _tc_combine(acc, deg, x_dst, Wl, bl, Wr):
  grid = (N // BM,)
  return pl.pallas_call(
      _tc_combine_body,
      grid=grid,
      in_specs=[
          pl.BlockSpec((BM, H), lambda i: (i, 0)),
          pl.BlockSpec((BM, H), lambda i: (i, 0)),
          pl.BlockSpec((BM, H), lambda i: (i, 0)),
          pl.BlockSpec((BM, D), lambda i: (i, 0)),
          pl.BlockSpec((D, D), lambda i: (0, 0)),
          pl.BlockSpec((1, D), lambda i: (0, 0)),
          pl.BlockSpec((D, D), lambda i: (0, 0)),
      ],
      out_specs=pl.BlockSpec((BM, D), lambda i: (i, 0)),
      out_shape=jax.ShapeDtypeStruct((N, D), jnp.float32),
  )(acc[0], acc[1], deg, x_dst, Wl, bl.reshape(1, D), Wr)


def kernel(x_sites, x_wells, edge_index_s2w, edge_index_w2s,
           Wl_s2w, bl_s2w, Wr_s2w, Wl_w2s, bl_w2s, Wr_w2s):
  xs_h = jnp.stack([x_sites[:, :H], x_sites[:, H:]])
  xw_h = jnp.stack([x_wells[:, :H], x_wells[:, H:]])
  src_s2w = edge_index_s2w[0].astype(jnp.int32)
  dst_s2w = edge_index_s2w[1].astype(jnp.int32)
  src_w2s = edge_index_w2s[0].astype(jnp.int32)
  dst_w2s = edge_index_w2s[1].astype(jnp.int32)

  acc_w, acc_s, deg_w, deg_s = _sc_segsum(xs_h, src_s2w, dst_s2w, xw_h,
                                          src_w2s, dst_w2s)

  z_wells = _tc_combine(acc_w, deg_w, x_wells, Wl_s2w, bl_s2w, Wr_s2w)
  z_sites = _tc_combine(acc_s, deg_s, x_sites, Wl_w2s, bl_w2s, Wr_w2s)
  return (z_sites, z_wells)


# speedup vs baseline: 5.9388x; 1.7647x over previous
import functools

import jax
import jax.numpy as jnp
from jax import lax
from jax.experimental import pallas as pl
from jax.experimental.pallas import tpu as pltpu
from jax.experimental.pallas import tpu_sc as plsc

N = 10000
D = 256
H = 128
E = 160000
K = 128
NCHUNK = E // K
NC = 2
NS = 16
R1 = 624
TAIL = N - NS * R1
NFULL = NCHUNK // NS - (1 if NCHUNK % NS else 0)
NFULL = 78
NTAIL = NCHUNK - NFULL * NS


def _sc_segsum(tab_a, eidx_a, tab_b, eidx_b):
  mesh = plsc.VectorSubcoreMesh(
      core_axis_name="c", subcore_axis_name="s", num_cores=NC,
      num_subcores=NS)

  @functools.partial(
      pl.kernel,
      out_type=[
          jax.ShapeDtypeStruct((NC, N, H), jnp.float32),
          jax.ShapeDtypeStruct((NC, N, H), jnp.float32),
          jax.ShapeDtypeStruct((N, H), jnp.float32),
          jax.ShapeDtypeStruct((N, H), jnp.float32),
      ],
      mesh=mesh,
      scratch_types=[
          pltpu.VMEM((2, K), jnp.int32),
          pltpu.VMEM((2, K), jnp.int32),
          pltpu.VMEM((K, H), jnp.float32),
          pltpu.VMEM((K, H), jnp.float32),
          pltpu.VMEM_SHARED((N, H), jnp.float32),
          pltpu.SemaphoreType.DMA,
          pltpu.SemaphoreType.DMA,
      ],
  )
  def seg_kernel(taba_h, eidxa_h, tabb_h, eidxb_h, zrows_h,
                 ones_hh, acca_out, accb_out, dega_out, degb_out,
                 idx_a, idx_b, rows_v, rows_w, acc_sh, sem_a, sem_b):
    c = lax.axis_index("c")
    s = lax.axis_index("s")
    r0 = s * R1
    last = s == NS - 1
    nfull = R1 // K
    rem = R1 - nfull * K

    def clear_accumulators():
      def clr(j, carry):
        pltpu.sync_copy(rows_v, acc_sh.at[pl.ds(r0 + j * K, K)])
        return carry

      lax.fori_loop(0, nfull, clr, 0)
      pltpu.sync_copy(rows_v.at[pl.ds(0, rem)],
                      acc_sh.at[pl.ds(r0 + R1 - rem, rem)])

      @pl.when(last)
      def _():
        pltpu.sync_copy(rows_v.at[pl.ds(0, TAIL)],
                        acc_sh.at[pl.ds(NS * R1, TAIL)])

    def run_edges(tab_h, eidx_h):
      def start(i, idx2, rows, sem2):
        pltpu.sync_copy(eidx_h.at[i * NS + s], idx2)
        pltpu.async_copy(tab_h.at[c].at[idx2.at[0]], rows, sem2)

      def finish(idx2, rows, sem2):
        pltpu.make_async_copy(tab_h.at[c].at[idx2.at[0]], rows,
                              sem2).wait()
        pltpu.sync_copy(rows, acc_sh.at[idx2.at[1]], add=True)

      start(0, idx_a, rows_v, sem_a)
      start(1, idx_b, rows_w, sem_b)

      def pair(i, carry):
        finish(idx_a, rows_v, sem_a)
        start(2 * i + 2, idx_a, rows_v, sem_a)
        finish(idx_b, rows_w, sem_b)
        start(2 * i + 3, idx_b, rows_w, sem_b)
        return carry

      lax.fori_loop(0, NFULL // 2 - 1, pair, 0)
      finish(idx_a, rows_v, sem_a)

      @pl.when(s < NTAIL)
      def _():
        start(NFULL, idx_a, rows_v, sem_a)

      finish(idx_b, rows_w, sem_b)

      @pl.when(s < NTAIL)
      def _():
        finish(idx_a, rows_v, sem_a)

    def write_back(get_dst):
      def wb(j, carry):
        pltpu.sync_copy(acc_sh.at[pl.ds(r0 + j * K, K)], rows_v)
        pltpu.sync_copy(rows_v, get_dst(r0 + j * K, K))
        return carry

      lax.fori_loop(0, nfull, wb, 0)
      pltpu.sync_copy(acc_sh.at[pl.ds(r0 + R1 - rem, rem)],
                      rows_v.at[pl.ds(0, rem)])
      pltpu.sync_copy(rows_v.at[pl.ds(0, rem)],
                      get_dst(r0 + R1 - rem, rem))

      @pl.when(last)
      def _():
        pltpu.sync_copy(acc_sh.at[pl.ds(NS * R1, TAIL)],
                        rows_v.at[pl.ds(rem, TAIL)])
        pltpu.sync_copy(rows_v.at[pl.ds(rem, TAIL)],
                        get_dst(NS * R1, TAIL))

    def run_deg(eidx_h):
      def start(i, idx2, sem2):
        pltpu.sync_copy(eidx_h.at[i * NS + s], idx2)
        pltpu.async_copy(rows_v, acc_sh.at[idx2.at[1]], sem2, add=True)

      def finish(idx2, sem2):
        pltpu.make_async_copy(rows_v, acc_sh.at[idx2.at[1]],
                              sem2).wait()

      start(0, idx_a, sem_a)
      start(1, idx_b, sem_b)

      def pair(i, carry):
        finish(idx_a, sem_a)
        start(2 * i + 2, idx_a, sem_a)
        finish(idx_b, sem_b)
        start(2 * i + 3, idx_b, sem_b)
        return carry

      lax.fori_loop(0, NFULL // 2 - 1, pair, 0)
      finish(idx_a, sem_a)

      @pl.when(s < NTAIL)
      def _():
        start(NFULL, idx_a, sem_a)

      finish(idx_b, sem_b)

      @pl.when(s < NTAIL)
      def _():
        finish(idx_a, sem_a)

    pltpu.sync_copy(zrows_h, rows_v)
    clear_accumulators()
    plsc.subcore_barrier()
    run_edges(taba_h, eidxa_h)
    plsc.subcore_barrier()
    write_back(lambda rr, nn: acca_out.at[c, pl.ds(rr, nn)])
    pltpu.sync_copy(zrows_h, rows_v)
    clear_accumulators()
    plsc.subcore_barrier()
    run_edges(tabb_h, eidxb_h)
    plsc.subcore_barrier()
    write_back(lambda rr, nn: accb_out.at[c, pl.ds(rr, nn)])
    pltpu.sync_copy(zrows_h, rows_v)
    clear_accumulators()
    plsc.subcore_barrier()
    pltpu.sync_copy(ones_hh, rows_v)

    @pl.when(c == 0)
    def _():
      run_deg(eidxa_h)

    @pl.when(c == 1)
    def _():
      run_deg(eidxb_h)

    plsc.subcore_barrier()

    @pl.when(c == 0)
    def _():
      write_back(lambda rr, nn: dega_out.at[pl.ds(rr, nn)])

    @pl.when(c == 1)
    def _():
      write_back(lambda rr, nn: degb_out.at[pl.ds(rr, nn)])

  return seg_kernel(tab_a, eidx_a, tab_b, eidx_b,
                    jnp.zeros((K, H), jnp.float32),
                    jnp.ones((K, H), jnp.float32))


BM = 1000


def _tc_combine_body(acc0_ref, acc1_ref, deg_ref, x_ref, wl_ref, bl_ref,
                     wr_ref, o_ref):
  a = jnp.concatenate([acc0_ref[...], acc1_ref[...]], axis=1)
  d = jnp.maximum(deg_ref[:, 0:1], 1.0)
  z = lax.dot(a, wl_ref[...], preferred_element_type=jnp.float32) / d
  z = z + bl_ref[...] + lax.dot(x_ref[...], wr_ref[...],
                                preferred_element_type=jnp.float32)
  o_ref[...] = z


def _tc_combine(acc, deg, x_dst, Wl, bl, Wr):
  grid = (N // BM,)
  return pl.pallas_call(
      _tc_combine_body,
      grid=grid,
      in_specs=[
          pl.BlockSpec((BM, H), lambda i: (i, 0)),
          pl.BlockSpec((BM, H), lambda i: (i, 0)),
          pl.BlockSpec((BM, H), lambda i: (i, 0)),
          pl.BlockSpec((BM, D), lambda i: (i, 0)),
          pl.BlockSpec((D, D), lambda i: (0, 0)),
          pl.BlockSpec((1, D), lambda i: (0, 0)),
          pl.BlockSpec((D, D), lambda i: (0, 0)),
      ],
      out_specs=pl.BlockSpec((BM, D), lambda i: (i, 0)),
      out_shape=jax.ShapeDtypeStruct((N, D), jnp.float32),
  )(acc[0], acc[1], deg, x_dst, Wl, bl.reshape(1, D), Wr)


def kernel(x_sites, x_wells, edge_index_s2w, edge_index_w2s,
           Wl_s2w, bl_s2w, Wr_s2w, Wl_w2s, bl_w2s, Wr_w2s):
  xs_h = jnp.stack([x_sites[:, :H], x_sites[:, H:]])
  xw_h = jnp.stack([x_wells[:, :H], x_wells[:, H:]])
  eidx_s2w = jnp.swapaxes(
      edge_index_s2w.astype(jnp.int32).reshape(2, NCHUNK, K), 0, 1)
  eidx_w2s = jnp.swapaxes(
      edge_index_w2s.astype(jnp.int32).reshape(2, NCHUNK, K), 0, 1)

  acc_w, acc_s, deg_w, deg_s = _sc_segsum(xs_h, eidx_s2w, xw_h, eidx_w2s)

  z_wells = _tc_combine(acc_w, deg_w, x_wells, Wl_s2w, bl_s2w, Wr_s2w)
  z_sites = _tc_combine(acc_s, deg_s, x_sites, Wl_w2s, bl_w2s, Wr_w2s)
  return (z_sites, z_wells)
